# baseline (speedup 1.0000x reference)
import functools

import jax
import jax.numpy as jnp
from jax import lax
from jax.experimental import pallas as pl
from jax.experimental.pallas import tpu as pltpu
from jax.experimental.pallas import tpu_sc as plsc

_VOCAB = 100000
_D = 50
_DP = 128
_SL = 56
_BATCH = 16384
_B = _BATCH * _D

_info = plsc.get_sparse_core_info()
_NC, _NS = _info.num_cores, _info.num_subcores
_NW = _NC * _NS
_BPW = _B // _NW
_RPW = _BPW // _D
_RH = _RPW // 2
_NB = 4
_G = _NB * _D
_NG = _BPW // _G
_NBUF = 3
_NGM = (_NG // _NBUF) * _NBUF
_L = 16

_mesh = plsc.VectorSubcoreMesh(core_axis_name="c", subcore_axis_name="s")


@functools.partial(
    pl.kernel,
    out_type=jax.ShapeDtypeStruct((_BATCH, _SL, _DP), jnp.float32),
    mesh=_mesh,
    scratch_types=[
        pltpu.VMEM((_RH, _D), jnp.int32),
        pltpu.VMEM((_NG, _G), jnp.int32),
        pltpu.VMEM((_NBUF, _G, _DP), jnp.float32),
        pltpu.SemaphoreType.DMA,
        pltpu.SemaphoreType.DMA,
        pltpu.SemaphoreType.DMA,
        pltpu.SemaphoreType.DMA,
        pltpu.SemaphoreType.DMA,
        pltpu.SemaphoreType.DMA,
    ],
    compiler_params=pltpu.CompilerParams(
        use_tc_tiling_on_sc=False, needs_layout_passes=False
    ),
)
def _gather(idx_hbm, table_hbm, out_hbm, idx_raw, idx_c, gbuf,
            gsem0, gsem1, gsem2, osem0, osem1, osem2):
    wid = lax.axis_index("s") * _NC + lax.axis_index("c")
    ebase = wid * _RPW
    gsems = (gsem0, gsem1, gsem2)
    osems = (osem0, osem1, osem2)

    lane = lax.iota(jnp.int32, _L)
    half_ids = _RH * _D

    for h in range(2):
        pltpu.sync_copy(idx_hbm.at[pl.ds(ebase + h * _RH, _RH)], idx_raw)

        def compact_idx(k, carry):
            e = k * _L + lane
            vals = plsc.load_gather(idx_raw, [e // _D, e % _D])
            p = h * half_ids + k * _L
            idx_c[p // _G, pl.ds(p % _G, _L)] = vals
            return carry

        lax.fori_loop(0, half_ids // _L, compact_idx, 0)

    def out_copies(buf, j, fn):
        for m in range(_NB):
            fn(
                gbuf.at[buf].at[pl.ds(m * _D, _D)],
                out_hbm.at[ebase + j * _NB + m, pl.ds(0, _D)],
                osems[buf],
            )

    def fire_gather(j, buf):
        pltpu.async_copy(table_hbm.at[idx_c.at[j]], gbuf.at[buf], gsems[buf])

    def wait_gather(j, buf):
        pltpu.make_async_copy(
            table_hbm.at[idx_c.at[j]], gbuf.at[buf], gsems[buf]
        ).wait()

    fire_gather(0, 0)
    fire_gather(1, 1)

    def outer(j3, carry):
        for b in range(_NBUF):
            j = j3 * _NBUF + b
            nb = (b + 2) % _NBUF

            @pl.when(j + 2 < _NG)
            def _fire():
                @pl.when(j >= 1)
                def _drain():
                    out_copies(
                        nb, j - 1,
                        lambda s, d, sem: pltpu.make_async_copy(s, d, sem).wait(),
                    )

                fire_gather(j + 2, nb)

            wait_gather(j, b)
            out_copies(b, j, pltpu.async_copy)
        return carry

    lax.fori_loop(0, _NGM // _NBUF, outer, 0)

    for j in range(_NGM, _NG):
        b = j % _NBUF
        wait_gather(j, b)
        out_copies(b, j, pltpu.async_copy)
    for j in range(_NG - _NBUF, _NG):
        out_copies(j % _NBUF, j,
                   lambda s, d, sem: pltpu.make_async_copy(s, d, sem).wait())


def kernel(indices, table):
    table_p = jnp.pad(table, ((0, 0), (0, _DP - _D)))
    out = _gather(indices.astype(jnp.int32), table_p)
    return out[:, :_D, :_D]
